# Initial kernel scaffold; baseline (speedup 1.0000x reference)
import jax
import jax.numpy as jnp
from jax.experimental import pallas as pl


def kernel(x, view_type_id, W1, b1, W2, b2, view_bias):
    raise NotImplementedError("write your pallas kernel here")


# speedup vs baseline: 3.0415x; 3.0415x over previous
import functools

import jax
import jax.numpy as jnp
from jax.experimental import pallas as pl


def _router_block(x_ref, w1_ref, b1_ref, w2_ref, b2_ref,
                  gate_ref, counts_ref, loss_ref, *, k, n_blocks, batch):
    i = pl.program_id(0)
    e = gate_ref.shape[-1]

    h = jax.lax.dot_general(
        x_ref[...], w1_ref[...], (((1,), (1,)), ((), ())),
        preferred_element_type=jnp.float32)
    h = h + b1_ref[...]
    h = 0.5 * h * (1.0 + jax.lax.erf(h * 0.7071067811865476))
    logits = jax.lax.dot_general(
        h, w2_ref[...], (((1,), (1,)), ((), ())),
        preferred_element_type=jnp.float32)
    logits = logits + b2_ref[...]

    idx = jax.lax.broadcasted_iota(jnp.int32, logits.shape, 1)
    work = logits
    selected = jnp.zeros(logits.shape, dtype=jnp.bool_)
    neg_inf = jnp.float32(-jnp.inf)
    for _ in range(k):
        m = jnp.max(work, axis=-1, keepdims=True)
        is_max = work == m
        first = jnp.min(jnp.where(is_max, idx, e), axis=-1, keepdims=True)
        pick = idx == first
        selected = jnp.logical_or(selected, pick)
        work = jnp.where(pick, neg_inf, work)

    max_val = jnp.max(logits, axis=-1, keepdims=True)
    exps = jnp.where(selected, jnp.exp(logits - max_val), 0.0)
    gate = exps / jnp.sum(exps, axis=-1, keepdims=True)
    gate_ref[...] = gate

    block_counts = jnp.sum(gate, axis=0, keepdims=True)

    @pl.when(i == 0)
    def _init():
        counts_ref[...] = block_counts

    @pl.when(i > 0)
    def _acc():
        counts_ref[...] = counts_ref[...] + block_counts

    @pl.when(i == n_blocks - 1)
    def _finish():
        counts = counts_ref[...]
        target = jnp.sum(counts) / e
        aux = jnp.mean((counts - target) ** 2)
        imp = counts / batch
        mean_imp = jnp.mean(imp)
        var1 = jnp.sum((imp - mean_imp) ** 2) / (e - 1)
        std1 = jnp.sqrt(var1)
        imp_loss = (std1 / (mean_imp + 1e-8)) ** 2
        loss_ref[...] = jnp.reshape(aux + 0.1 * imp_loss, (1, 1))


def kernel(x, view_type_id, W1, b1, W2, b2, view_bias):
    batch, d = x.shape
    h_dim = W1.shape[0]
    e = W2.shape[0]
    k = 8

    bm = 512 if batch % 512 == 0 else batch
    n_blocks = batch // bm

    b1r = b1.reshape(1, h_dim)
    b2r = (b2 + view_bias[view_type_id]).reshape(1, e)

    body = functools.partial(_router_block, k=k, n_blocks=n_blocks,
                             batch=batch)
    gate, _counts, loss = pl.pallas_call(
        body,
        grid=(n_blocks,),
        in_specs=[
            pl.BlockSpec((bm, d), lambda i: (i, 0)),
            pl.BlockSpec((h_dim, d), lambda i: (0, 0)),
            pl.BlockSpec((1, h_dim), lambda i: (0, 0)),
            pl.BlockSpec((e, h_dim), lambda i: (0, 0)),
            pl.BlockSpec((1, e), lambda i: (0, 0)),
        ],
        out_specs=[
            pl.BlockSpec((bm, e), lambda i: (i, 0)),
            pl.BlockSpec((1, e), lambda i: (0, 0)),
            pl.BlockSpec((1, 1), lambda i: (0, 0)),
        ],
        out_shape=[
            jax.ShapeDtypeStruct((batch, e), jnp.float32),
            jax.ShapeDtypeStruct((1, e), jnp.float32),
            jax.ShapeDtypeStruct((1, 1), jnp.float32),
        ],
    )(x, W1, b1r, W2, b2r)
    return gate, loss[0, 0]


# speedup vs baseline: 3.4404x; 1.1311x over previous
import functools

import jax
import jax.numpy as jnp
from jax.experimental import pallas as pl


def _router_block(x_ref, w1_ref, b1_ref, w2_ref, b2_ref,
                  gate_ref, counts_ref, loss_ref, *, k, n_blocks, batch):
    i = pl.program_id(0)
    e = gate_ref.shape[-1]

    h = jax.lax.dot_general(
        x_ref[...], w1_ref[...], (((1,), (1,)), ((), ())),
        preferred_element_type=jnp.float32)
    h = h + b1_ref[...]
    h = 0.5 * h * (1.0 + jax.lax.erf(h * 0.7071067811865476))
    logits = jax.lax.dot_general(
        h, w2_ref[...], (((1,), (1,)), ((), ())),
        preferred_element_type=jnp.float32)
    logits = logits + b2_ref[...]

    work = logits
    neg_inf = jnp.float32(-jnp.inf)
    m = None
    for _ in range(k):
        m = jnp.max(work, axis=-1, keepdims=True)
        work = jnp.where(work >= m, neg_inf, work)
    selected = logits >= m

    max_val = jnp.max(logits, axis=-1, keepdims=True)
    exps = jnp.where(selected, jnp.exp(logits - max_val), 0.0)
    gate = exps / jnp.sum(exps, axis=-1, keepdims=True)
    gate_ref[...] = gate

    block_counts = jnp.sum(gate, axis=0, keepdims=True)

    @pl.when(i == 0)
    def _init():
        counts_ref[...] = block_counts

    @pl.when(i > 0)
    def _acc():
        counts_ref[...] = counts_ref[...] + block_counts

    @pl.when(i == n_blocks - 1)
    def _finish():
        counts = counts_ref[...]
        target = jnp.sum(counts) / e
        aux = jnp.mean((counts - target) ** 2)
        imp = counts / batch
        mean_imp = jnp.mean(imp)
        var1 = jnp.sum((imp - mean_imp) ** 2) / (e - 1)
        std1 = jnp.sqrt(var1)
        imp_loss = (std1 / (mean_imp + 1e-8)) ** 2
        loss_ref[...] = jnp.reshape(aux + 0.1 * imp_loss, (1, 1))


def kernel(x, view_type_id, W1, b1, W2, b2, view_bias):
    batch, d = x.shape
    h_dim = W1.shape[0]
    e = W2.shape[0]
    k = 8

    bm = 512 if batch % 512 == 0 else batch
    n_blocks = batch // bm

    b1r = b1.reshape(1, h_dim)
    b2r = (b2 + view_bias[view_type_id]).reshape(1, e)

    body = functools.partial(_router_block, k=k, n_blocks=n_blocks,
                             batch=batch)
    gate, _counts, loss = pl.pallas_call(
        body,
        grid=(n_blocks,),
        in_specs=[
            pl.BlockSpec((bm, d), lambda i: (i, 0)),
            pl.BlockSpec((h_dim, d), lambda i: (0, 0)),
            pl.BlockSpec((1, h_dim), lambda i: (0, 0)),
            pl.BlockSpec((e, h_dim), lambda i: (0, 0)),
            pl.BlockSpec((1, e), lambda i: (0, 0)),
        ],
        out_specs=[
            pl.BlockSpec((bm, e), lambda i: (i, 0)),
            pl.BlockSpec((1, e), lambda i: (0, 0)),
            pl.BlockSpec((1, 1), lambda i: (0, 0)),
        ],
        out_shape=[
            jax.ShapeDtypeStruct((batch, e), jnp.float32),
            jax.ShapeDtypeStruct((1, e), jnp.float32),
            jax.ShapeDtypeStruct((1, 1), jnp.float32),
        ],
    )(x, W1, b1r, W2, b2r)
    return gate, loss[0, 0]


# speedup vs baseline: 3.5466x; 1.0309x over previous
import functools

import jax
import jax.numpy as jnp
from jax.experimental import pallas as pl
from jax.experimental.pallas import tpu as pltpu


def _router_block(x_ref, w1_ref, b1_ref, w2_ref, b2_ref,
                  gate_ref, counts_ref, loss_ref, logits_scr,
                  *, k, n_blocks, batch):
    i = pl.program_id(0)
    e = gate_ref.shape[-1]

    logits = logits_scr[...]

    xb = x_ref[...]
    h_dim = w1_ref.shape[0]
    hc = min(1024, h_dim)
    new_logits = jnp.zeros((xb.shape[0], e), jnp.float32)
    for c in range(0, h_dim, hc):
        hcnk = jax.lax.dot_general(
            xb, w1_ref[pl.ds(c, hc), :], (((1,), (1,)), ((), ())),
            preferred_element_type=jnp.float32)
        hcnk = hcnk + b1_ref[:, pl.ds(c, hc)]
        hcnk = 0.5 * hcnk * (1.0 + jax.lax.erf(hcnk * 0.7071067811865476))
        new_logits = new_logits + jax.lax.dot_general(
            hcnk, w2_ref[:, pl.ds(c, hc)], (((1,), (1,)), ((), ())),
            preferred_element_type=jnp.float32)
    logits_scr[...] = new_logits + b2_ref[...]

    work = logits
    neg_inf = jnp.float32(-jnp.inf)
    m = None
    for _ in range(k):
        m = jnp.max(work, axis=-1, keepdims=True)
        work = jnp.where(work >= m, neg_inf, work)
    selected = logits >= m

    max_val = jnp.max(logits, axis=-1, keepdims=True)
    exps = jnp.where(selected, jnp.exp(logits - max_val), 0.0)
    gate = exps / jnp.sum(exps, axis=-1, keepdims=True)
    gate_ref[...] = gate

    block_counts = jnp.sum(gate, axis=0, keepdims=True)

    @pl.when(i == 1)
    def _init():
        counts_ref[...] = block_counts

    @pl.when(i > 1)
    def _acc():
        counts_ref[...] = counts_ref[...] + block_counts

    @pl.when(i == n_blocks)
    def _finish():
        counts = counts_ref[...]
        target = jnp.sum(counts) / e
        aux = jnp.mean((counts - target) ** 2)
        imp = counts / batch
        mean_imp = jnp.mean(imp)
        var1 = jnp.sum((imp - mean_imp) ** 2) / (e - 1)
        std1 = jnp.sqrt(var1)
        imp_loss = (std1 / (mean_imp + 1e-8)) ** 2
        loss_ref[...] = jnp.reshape(aux + 0.1 * imp_loss, (1, 1))


def kernel(x, view_type_id, W1, b1, W2, b2, view_bias):
    batch, d = x.shape
    h_dim = W1.shape[0]
    e = W2.shape[0]
    k = 8

    bm = 512 if batch % 512 == 0 else batch
    n_blocks = batch // bm

    b1r = b1.reshape(1, h_dim)
    b2r = (b2 + view_bias[view_type_id]).reshape(1, e)

    body = functools.partial(_router_block, k=k, n_blocks=n_blocks,
                             batch=batch)
    last = n_blocks - 1
    gate, _counts, loss = pl.pallas_call(
        body,
        grid=(n_blocks + 1,),
        in_specs=[
            pl.BlockSpec((bm, d), lambda i: (jnp.minimum(i, last), 0)),
            pl.BlockSpec((h_dim, d), lambda i: (0, 0)),
            pl.BlockSpec((1, h_dim), lambda i: (0, 0)),
            pl.BlockSpec((e, h_dim), lambda i: (0, 0)),
            pl.BlockSpec((1, e), lambda i: (0, 0)),
        ],
        out_specs=[
            pl.BlockSpec((bm, e), lambda i: (jnp.maximum(i - 1, 0), 0)),
            pl.BlockSpec((1, e), lambda i: (0, 0)),
            pl.BlockSpec((1, 1), lambda i: (0, 0)),
        ],
        out_shape=[
            jax.ShapeDtypeStruct((batch, e), jnp.float32),
            jax.ShapeDtypeStruct((1, e), jnp.float32),
            jax.ShapeDtypeStruct((1, 1), jnp.float32),
        ],
        scratch_shapes=[pltpu.VMEM((bm, e), jnp.float32)],
    )(x, W1, b1r, W2, b2r)
    return gate, loss[0, 0]


# speedup vs baseline: 3.5600x; 1.0038x over previous
import functools

import jax
import jax.numpy as jnp
from jax.experimental import pallas as pl
from jax.experimental.pallas import tpu as pltpu


def _router_block(vid_ref, x_ref, w1_ref, b1_ref, w2_ref, b2_ref, vb_ref,
                  gate_ref, counts_ref, loss_ref, logits_scr,
                  *, k, n_blocks, batch):
    i = pl.program_id(0)
    e = gate_ref.shape[-1]

    logits = logits_scr[...]

    xb = x_ref[...]
    h_dim = w1_ref.shape[0]
    hc = min(1024, h_dim)
    new_logits = jnp.zeros((xb.shape[0], e), jnp.float32)
    for c in range(0, h_dim, hc):
        hcnk = jax.lax.dot_general(
            xb, w1_ref[pl.ds(c, hc), :], (((1,), (1,)), ((), ())),
            preferred_element_type=jnp.float32)
        hcnk = hcnk + b1_ref[:, pl.ds(c, hc)]
        hcnk = 0.5 * hcnk * (1.0 + jax.lax.erf(hcnk * 0.7071067811865476))
        new_logits = new_logits + jax.lax.dot_general(
            hcnk, w2_ref[:, pl.ds(c, hc)], (((1,), (1,)), ((), ())),
            preferred_element_type=jnp.float32)
    bias = b2_ref[...] + vb_ref[pl.ds(vid_ref[0], 1), :]
    logits_scr[...] = new_logits + bias

    work = logits
    neg_inf = jnp.float32(-jnp.inf)
    m = None
    for _ in range(k):
        m = jnp.max(work, axis=-1, keepdims=True)
        work = jnp.where(work >= m, neg_inf, work)
    selected = logits >= m

    max_val = jnp.max(logits, axis=-1, keepdims=True)
    exps = jnp.where(selected, jnp.exp(logits - max_val), 0.0)
    gate = exps / jnp.sum(exps, axis=-1, keepdims=True)
    gate_ref[...] = gate

    block_counts = jnp.sum(gate, axis=0, keepdims=True)

    @pl.when(i == 1)
    def _init():
        counts_ref[...] = block_counts

    @pl.when(i > 1)
    def _acc():
        counts_ref[...] = counts_ref[...] + block_counts

    @pl.when(i == n_blocks)
    def _finish():
        counts = counts_ref[...]
        target = jnp.sum(counts) / e
        aux = jnp.mean((counts - target) ** 2)
        imp = counts / batch
        mean_imp = jnp.mean(imp)
        var1 = jnp.sum((imp - mean_imp) ** 2) / (e - 1)
        std1 = jnp.sqrt(var1)
        imp_loss = (std1 / (mean_imp + 1e-8)) ** 2
        loss_ref[...] = jnp.reshape(aux + 0.1 * imp_loss, (1, 1))


def kernel(x, view_type_id, W1, b1, W2, b2, view_bias):
    batch, d = x.shape
    h_dim = W1.shape[0]
    e = W2.shape[0]
    k = 8

    bm = 512 if batch % 512 == 0 else batch
    n_blocks = batch // bm

    b1r = b1.reshape(1, h_dim)
    b2r = b2.reshape(1, e)
    vid = jnp.asarray(view_type_id, jnp.int32).reshape(1)

    body = functools.partial(_router_block, k=k, n_blocks=n_blocks,
                             batch=batch)
    last = n_blocks - 1
    gate, _counts, loss = pl.pallas_call(
        body,
        grid=(n_blocks + 1,),
        in_specs=[
            pl.BlockSpec(memory_space=pltpu.SMEM),
            pl.BlockSpec((bm, d), lambda i: (jnp.minimum(i, last), 0)),
            pl.BlockSpec((h_dim, d), lambda i: (0, 0)),
            pl.BlockSpec((1, h_dim), lambda i: (0, 0)),
            pl.BlockSpec((e, h_dim), lambda i: (0, 0)),
            pl.BlockSpec((1, e), lambda i: (0, 0)),
            pl.BlockSpec((2, e), lambda i: (0, 0)),
        ],
        out_specs=[
            pl.BlockSpec((bm, e), lambda i: (jnp.maximum(i - 1, 0), 0)),
            pl.BlockSpec((1, e), lambda i: (0, 0)),
            pl.BlockSpec((1, 1), lambda i: (0, 0)),
        ],
        out_shape=[
            jax.ShapeDtypeStruct((batch, e), jnp.float32),
            jax.ShapeDtypeStruct((1, e), jnp.float32),
            jax.ShapeDtypeStruct((1, 1), jnp.float32),
        ],
        scratch_shapes=[pltpu.VMEM((bm, e), jnp.float32)],
    )(vid, x, W1, b1r, W2, b2r, view_bias)
    return gate, loss.reshape(())


# speedup vs baseline: 3.5626x; 1.0007x over previous
import functools

import jax
import jax.numpy as jnp
from jax.experimental import pallas as pl
from jax.experimental.pallas import tpu as pltpu


def _router_block(vid_ref, x_ref, w1_ref, b1_ref, w2_ref, b2_ref, vb_ref,
                  gate_ref, counts_ref, loss_ref, logits_scr,
                  *, k, n_blocks, batch):
    i = pl.program_id(0)
    e = gate_ref.shape[-1]

    logits = logits_scr[...]

    xb = x_ref[...]
    h_dim = w1_ref.shape[0]
    hc = min(1024, h_dim)
    new_logits = None
    for c in range(0, h_dim, hc):
        hcnk = jax.lax.dot_general(
            xb, w1_ref[pl.ds(c, hc), :], (((1,), (1,)), ((), ())),
            preferred_element_type=jnp.float32)
        hcnk = hcnk + b1_ref[:, pl.ds(c, hc)]
        hcnk = 0.5 * hcnk * (1.0 + jax.lax.erf(hcnk * 0.7071067811865476))
        part = jax.lax.dot_general(
            hcnk, w2_ref[:, pl.ds(c, hc)], (((1,), (1,)), ((), ())),
            preferred_element_type=jnp.float32)
        new_logits = part if new_logits is None else new_logits + part
    bias = b2_ref[...] + vb_ref[pl.ds(vid_ref[0], 1), :]
    logits_scr[...] = new_logits + bias

    work = logits
    neg_inf = jnp.float32(-jnp.inf)
    m = max_val = None
    for step in range(k):
        m = jnp.max(work, axis=-1, keepdims=True)
        if step == 0:
            max_val = m
        work = jnp.where(work >= m, neg_inf, work)
    selected = logits >= m

    exps = jnp.where(selected, jnp.exp(logits - max_val), 0.0)
    gate = exps / jnp.sum(exps, axis=-1, keepdims=True)
    gate_ref[...] = gate

    block_counts = jnp.sum(gate, axis=0, keepdims=True)

    @pl.when(i == 1)
    def _init():
        counts_ref[...] = block_counts

    @pl.when(i > 1)
    def _acc():
        counts_ref[...] = counts_ref[...] + block_counts

    @pl.when(i == n_blocks)
    def _finish():
        counts = counts_ref[...]
        target = jnp.sum(counts) / e
        aux = jnp.mean((counts - target) ** 2)
        imp = counts / batch
        mean_imp = jnp.mean(imp)
        var1 = jnp.sum((imp - mean_imp) ** 2) / (e - 1)
        std1 = jnp.sqrt(var1)
        imp_loss = (std1 / (mean_imp + 1e-8)) ** 2
        loss_ref[...] = jnp.reshape(aux + 0.1 * imp_loss, (1, 1))


def kernel(x, view_type_id, W1, b1, W2, b2, view_bias):
    batch, d = x.shape
    h_dim = W1.shape[0]
    e = W2.shape[0]
    k = 8

    bm = 512 if batch % 512 == 0 else batch
    n_blocks = batch // bm

    b1r = b1.reshape(1, h_dim)
    b2r = b2.reshape(1, e)
    vid = jnp.asarray(view_type_id, jnp.int32).reshape(1)

    body = functools.partial(_router_block, k=k, n_blocks=n_blocks,
                             batch=batch)
    last = n_blocks - 1
    gate, _counts, loss = pl.pallas_call(
        body,
        grid=(n_blocks + 1,),
        in_specs=[
            pl.BlockSpec(memory_space=pltpu.SMEM),
            pl.BlockSpec((bm, d), lambda i: (jnp.minimum(i, last), 0)),
            pl.BlockSpec((h_dim, d), lambda i: (0, 0)),
            pl.BlockSpec((1, h_dim), lambda i: (0, 0)),
            pl.BlockSpec((e, h_dim), lambda i: (0, 0)),
            pl.BlockSpec((1, e), lambda i: (0, 0)),
            pl.BlockSpec((2, e), lambda i: (0, 0)),
        ],
        out_specs=[
            pl.BlockSpec((bm, e), lambda i: (jnp.maximum(i - 1, 0), 0)),
            pl.BlockSpec((1, e), lambda i: (0, 0)),
            pl.BlockSpec((1, 1), lambda i: (0, 0)),
        ],
        out_shape=[
            jax.ShapeDtypeStruct((batch, e), jnp.float32),
            jax.ShapeDtypeStruct((1, e), jnp.float32),
            jax.ShapeDtypeStruct((1, 1), jnp.float32),
        ],
        scratch_shapes=[pltpu.VMEM((bm, e), jnp.float32)],
    )(vid, x, W1, b1r, W2, b2r, view_bias)
    return gate, loss.reshape(())
